# baseline (speedup 1.0000x reference)
import jax
import jax.numpy as jnp
from jax import lax
from jax.experimental import pallas as pl
from jax.experimental.pallas import tpu as pltpu
from jax.experimental.pallas import tpu_sc as plsc

_NC = 2
_NS = 16
_NW = _NC * _NS
_C = 32


def _sc_body(seq_len, batch, d_model, x_hbm, t_hbm, o_hbm,
             xbuf0, xbuf1, tbuf, lsem0, lsem1, ssem0, ssem1):
    wid = lax.axis_index("s") * _NC + lax.axis_index("c")
    rows_per_worker = seq_len // _NW
    nchunk = rows_per_worker // _C
    words = _C * d_model
    tbase = wid * rows_per_worker

    xbufs = (xbuf0, xbuf1)
    lsems = (lsem0, lsem1)
    ssems = (ssem0, ssem1)
    n_iters = nchunk * batch

    def x_off(it):
        j, b = divmod(it, batch)
        return (b * seq_len + tbase + j * _C) * d_model

    loads = [None, None]
    stores = [None, None]
    loads[0] = pltpu.async_copy(
        x_hbm.at[pl.ds(x_off(0), words)], xbufs[0], lsems[0])

    for j in range(nchunk):
        pltpu.sync_copy(
            t_hbm.at[pl.ds((tbase + j * _C) * d_model, words)], tbuf)
        for b in range(batch):
            it = j * batch + b
            cur = it % 2
            nxt = 1 - cur
            if it + 1 < n_iters:
                if stores[nxt] is not None:
                    stores[nxt].wait()
                    stores[nxt] = None
                loads[nxt] = pltpu.async_copy(
                    x_hbm.at[pl.ds(x_off(it + 1), words)], xbufs[nxt],
                    lsems[nxt])
            loads[cur].wait()
            loads[cur] = None

            buf = xbufs[cur]

            def add_body(i, carry):
                base = i * 64
                for u in range(4):
                    s = pl.ds(base + u * 16, 16)
                    buf[s] = buf[s] + tbuf[s]
                return carry

            lax.fori_loop(0, words // 64, add_body, 0)

            stores[cur] = pltpu.async_copy(
                buf, o_hbm.at[pl.ds(x_off(it), words)], ssems[cur])

    for k in range(2):
        if stores[k] is not None:
            stores[k].wait()


def kernel(x, pos_table):
    batch, seq_len, d_model = x.shape
    words = _C * d_model
    xf = x.reshape(-1)
    tf = pos_table.reshape(-1)

    import functools
    body = functools.partial(_sc_body, seq_len, batch, d_model)
    out = pl.kernel(
        body,
        out_type=jax.ShapeDtypeStruct((batch * seq_len * d_model,), x.dtype),
        mesh=plsc.VectorSubcoreMesh(core_axis_name="c", subcore_axis_name="s"),
        scratch_types=[
            pltpu.VMEM((words,), jnp.float32),
            pltpu.VMEM((words,), jnp.float32),
            pltpu.VMEM((words,), jnp.float32),
            pltpu.SemaphoreType.DMA,
            pltpu.SemaphoreType.DMA,
            pltpu.SemaphoreType.DMA,
            pltpu.SemaphoreType.DMA,
        ],
    )(xf, tf)
    return out.reshape(batch, seq_len, d_model)
